# baseline (device time: 35153 ns/iter reference)
import jax
import jax.numpy as jnp
from jax import lax
from jax.experimental import pallas as pl
from jax.experimental.pallas import tpu as pltpu

N_DEV = 16
B, SQ, SKV = 2, 256, 256
HQ_TOT, DH = 64, 64
H_LOC = HQ_TOT // N_DEV
BLK = 64
D_MODEL = 512
D_HEADS = H_LOC * DH
ROWS = B * SQ
SEG = ROWS // N_DEV
CHUNK = 128
SEG_PER_CHUNK = CHUNK // SEG


def _block_mask(q0, nq, nk):
    rb = q0 + lax.broadcasted_iota(jnp.int32, (nq * BLK, nk * BLK), 0) // BLK
    cb = lax.broadcasted_iota(jnp.int32, (nq * BLK, nk * BLK), 1) // BLK
    return cb <= rb


def kernel(x, Wq, K_ext, V_ext, Wo):
    K2 = K_ext.reshape(B, SKV, D_HEADS)
    V2 = V_ext.reshape(B, SKV, D_HEADS)

    def body(x_ref, wq_ref, k_ref, v_ref, wo_ref, out_ref,
             pbf_ref, stage_ref, gbuf_ref, allout_ref,
             wq_vmem, wo_vmem, wdma_sems,
             send1, recv1, send2, recv2):
        my = lax.axis_index("i")

        wq_dma = pltpu.make_async_copy(
            wq_ref.at[:, pl.ds(my * D_HEADS, D_HEADS)], wq_vmem,
            wdma_sems.at[0])
        wo_dma = pltpu.make_async_copy(
            wo_ref.at[pl.ds(my * D_HEADS, D_HEADS), :], wo_vmem,
            wdma_sems.at[1])
        wq_dma.start()
        wo_dma.start()

        xx = x_ref[...].reshape(ROWS, D_MODEL).astype(jnp.bfloat16)
        wq_dma.wait()
        wq = wq_vmem[...].astype(jnp.bfloat16)
        q_all = jnp.dot(xx, wq, preferred_element_type=jnp.float32)
        wo_dma.wait()
        wo = wo_vmem[...].astype(jnp.bfloat16)

        barrier = pltpu.get_barrier_semaphore()
        for d in range(1, N_DEV):
            pl.semaphore_signal(
                barrier, inc=1,
                device_id=((my + d) % N_DEV,),
                device_id_type=pl.DeviceIdType.MESH,
            )
        pl.semaphore_wait(barrier, N_DEV - 1)

        masks = {2: _block_mask(0, 2, 2), 4: _block_mask(2, 2, 4)}
        for c in range(ROWS // CHUNK):
            b, half = divmod(c, 2)
            nk = 2 if half == 0 else 4
            kn = nk * BLK
            mask = masks[nk]
            r0 = c * CHUNK
            kb = k_ref[b].astype(jnp.bfloat16)
            vb = v_ref[b].astype(jnp.bfloat16)
            qc = q_all[r0:r0 + CHUNK]
            heads = []
            for h in range(H_LOC):
                qh = qc[:, h * DH:(h + 1) * DH].astype(jnp.bfloat16)
                kh = kb[:kn, h * DH:(h + 1) * DH]
                vh = vb[:kn, h * DH:(h + 1) * DH]
                s = lax.dot_general(
                    qh, kh, (((1,), (1,)), ((), ())),
                    preferred_element_type=jnp.float32,
                ) * 0.125
                w = jnp.where(mask, jnp.exp(s), 0.0)
                w = w / jnp.sum(w, axis=-1, keepdims=True)
                heads.append(jnp.dot(
                    w.astype(jnp.bfloat16), vh,
                    preferred_element_type=jnp.float32,
                ))
            ctx = jnp.concatenate(heads, axis=1).astype(jnp.bfloat16)
            part = jnp.dot(ctx, wo, preferred_element_type=jnp.float32)
            pbf_ref[pl.ds(r0, CHUNK), :] = part.astype(jnp.bfloat16)

            for j in range(SEG_PER_CHUNK):
                p = c * SEG_PER_CHUNK + j

                @pl.when(p != my)
                def _send():
                    pltpu.make_async_remote_copy(
                        src_ref=pbf_ref.at[pl.ds(p * SEG, SEG)],
                        dst_ref=stage_ref.at[pl.ds(my * SEG, SEG)],
                        send_sem=send1.at[p],
                        recv_sem=recv1.at[my],
                        device_id=(p,),
                        device_id_type=pl.DeviceIdType.MESH,
                    ).start()

                @pl.when(p == my)
                def _local():
                    stage_ref[pl.ds(p * SEG, SEG), :] = \
                        pbf_ref[pl.ds(p * SEG, SEG), :]

        for d in range(1, N_DEV):
            q_src = (my + d) % N_DEV
            pltpu.make_async_remote_copy(
                src_ref=pbf_ref.at[pl.ds(0, SEG)],
                dst_ref=stage_ref.at[pl.ds(q_src * SEG, SEG)],
                send_sem=send1.at[q_src],
                recv_sem=recv1.at[q_src],
                device_id=(q_src,),
                device_id_type=pl.DeviceIdType.MESH,
            ).wait_recv()
        for d in range(1, N_DEV):
            p = (my + d) % N_DEV
            pltpu.make_async_remote_copy(
                src_ref=pbf_ref.at[pl.ds(0, SEG)],
                dst_ref=stage_ref.at[pl.ds(p * SEG, SEG)],
                send_sem=send1.at[p],
                recv_sem=recv1.at[p],
                device_id=(p,),
                device_id_type=pl.DeviceIdType.MESH,
            ).wait_send()

        s = stage_ref[...].astype(jnp.float32)
        seg_sum = s.reshape(N_DEV, SEG, D_MODEL).sum(axis=0)
        gbuf_ref[...] = seg_sum.astype(jnp.bfloat16)

        sends2 = []
        for d in range(1, N_DEV):
            p = (my + d) % N_DEV
            r = pltpu.make_async_remote_copy(
                src_ref=gbuf_ref,
                dst_ref=allout_ref.at[pl.ds(my * SEG, SEG)],
                send_sem=send2.at[p],
                recv_sem=recv2.at[my],
                device_id=(p,),
                device_id_type=pl.DeviceIdType.MESH,
            )
            r.start()
            sends2.append(r)
        allout_ref[pl.ds(my * SEG, SEG), :] = gbuf_ref[...]
        for d in range(1, N_DEV):
            q_src = (my + d) % N_DEV
            pltpu.make_async_remote_copy(
                src_ref=gbuf_ref,
                dst_ref=allout_ref.at[pl.ds(q_src * SEG, SEG)],
                send_sem=send2.at[q_src],
                recv_sem=recv2.at[q_src],
                device_id=(q_src,),
                device_id_type=pl.DeviceIdType.MESH,
            ).wait_recv()
        for r in sends2:
            r.wait_send()

        out_ref[...] = allout_ref[...].astype(jnp.float32).reshape(
            B, SQ, D_MODEL)

    return pl.pallas_call(
        body,
        out_shape=jax.ShapeDtypeStruct((B, SQ, D_MODEL), jnp.float32),
        in_specs=[
            pl.BlockSpec(memory_space=pltpu.VMEM),
            pl.BlockSpec(memory_space=pl.ANY),
            pl.BlockSpec(memory_space=pltpu.VMEM),
            pl.BlockSpec(memory_space=pltpu.VMEM),
            pl.BlockSpec(memory_space=pl.ANY),
        ],
        out_specs=pl.BlockSpec(memory_space=pltpu.VMEM),
        scratch_shapes=[
            pltpu.VMEM((ROWS, D_MODEL), jnp.bfloat16),
            pltpu.VMEM((ROWS, D_MODEL), jnp.bfloat16),
            pltpu.VMEM((SEG, D_MODEL), jnp.bfloat16),
            pltpu.VMEM((ROWS, D_MODEL), jnp.bfloat16),
            pltpu.VMEM((D_MODEL, D_HEADS), jnp.float32),
            pltpu.VMEM((D_HEADS, D_MODEL), jnp.float32),
            pltpu.SemaphoreType.DMA((2,)),
            pltpu.SemaphoreType.DMA((N_DEV,)),
            pltpu.SemaphoreType.DMA((N_DEV,)),
            pltpu.SemaphoreType.DMA((N_DEV,)),
            pltpu.SemaphoreType.DMA((N_DEV,)),
        ],
        compiler_params=pltpu.CompilerParams(collective_id=0),
    )(x, Wq, K2, V2, Wo)
